# baseline (speedup 1.0000x reference)
import jax
import jax.numpy as jnp
from jax import lax
from jax.experimental import pallas as pl
from jax.experimental.pallas import tpu as pltpu
from jax.experimental.pallas import tpu_sc as plsc

NUM_CORES = 2
NUM_SUBCORES = 16
LANES = 16
NW = NUM_CORES * NUM_SUBCORES
IDX_CHUNK = 128

NROWS = 1000000
R = 64
NBLK = NROWS // IDX_CHUNK
BLK_PER_W = NBLK // NW
BLK_LEFT = NBLK - BLK_PER_W * NW
TAIL = NROWS - NBLK * IDX_CHUNK
NPAIR = NROWS // 2


def _wid():
    return lax.axis_index("s") * NUM_CORES + lax.axis_index("c")


def _bias_body(u_hbm, it_hbm, bu_hbm, bi_hbm, mu_hbm, out_hbm,
               u_v, it_v, buv, biv, mu_v, out_v, sem_bu, sem_bi):
    bpw = out_v.shape[0]
    n_chunks = u_v.shape[0]
    wid = _wid()
    base = wid * bpw

    pltpu.sync_copy(u_hbm.at[pl.ds(wid * n_chunks, n_chunks)], u_v)
    pltpu.sync_copy(it_hbm.at[pl.ds(wid * n_chunks, n_chunks)], it_v)
    pltpu.sync_copy(mu_hbm, mu_v)

    copies = []
    for k in range(n_chunks):
        dst = pl.ds(k * IDX_CHUNK, IDX_CHUNK)
        copies.append(pltpu.async_copy(bu_hbm.at[u_v.at[k]], buv.at[dst], sem_bu))
        copies.append(pltpu.async_copy(bi_hbm.at[it_v.at[k]], biv.at[dst], sem_bi))
    for c in copies:
        c.wait()

    def group(g, carry):
        s = pl.ds(g * LANES, LANES)
        out_v[s] = mu_v[...] + buv[s] + biv[s]
        return carry

    lax.fori_loop(0, bpw // LANES, group, 0)
    pltpu.sync_copy(out_v, out_hbm.at[pl.ds(base, bpw)])


def _dot_body(u_hbm, it_hbm, abf_hbm, bbf_hbm, bs_hbm, out_hbm,
              u_v, it_v, bs_v, pa, pb, arows, brows, out_v, sem_a, sem_b):
    bpw = out_v.shape[0]
    n_chunks = bpw // IDX_CHUNK
    wid = _wid()
    base = wid * bpw

    pltpu.sync_copy(u_hbm.at[pl.ds(base, bpw)], u_v)
    pltpu.sync_copy(it_hbm.at[pl.ds(base, bpw)], it_v)
    pltpu.sync_copy(bs_hbm.at[pl.ds(base, bpw)], bs_v)

    def mkidx(g, carry):
        k = g // (IDX_CHUNK // LANES)
        off = (g % (IDX_CHUNK // LANES)) * LANES
        pa[k, pl.ds(off, LANES)] = u_v[pl.ds(g * LANES, LANES)] >> 1
        pb[k, pl.ds(off, LANES)] = it_v[pl.ds(g * LANES, LANES)] >> 1
        return carry

    lax.fori_loop(0, bpw // LANES, mkidx, 0)

    iota = lax.iota(jnp.int32, LANES)
    rows_per_round = arows.shape[0]
    chunks_per_round = rows_per_round // IDX_CHUNK

    for rnd in range(bpw // rows_per_round):
        copies = []
        for kk in range(chunks_per_round):
            k = rnd * chunks_per_round + kk
            dst = pl.ds(kk * IDX_CHUNK, IDX_CHUNK)
            copies.append(
                pltpu.async_copy(abf_hbm.at[pa.at[k]], arows.at[dst], sem_a))
            copies.append(
                pltpu.async_copy(bbf_hbm.at[pb.at[k]], brows.at[dst], sem_b))
        for c in copies:
            c.wait()

        def group(g, carry, rnd=rnd):
            goff = rnd * rows_per_round + g * LANES
            uvec = u_v[pl.ds(goff, LANES)]
            ivec = it_v[pl.ds(goff, LANES)]
            acc = bs_v[pl.ds(goff, LANES)]
            for l in range(LANES):
                n = g * LANES + l
                pu = uvec[l] & 1
                pv = ivec[l] & 1
                q = None
                for c in range(R // LANES):
                    ec = arows[n, pl.ds(c * LANES, LANES)]
                    oc = arows[n, pl.ds(R + c * LANES, LANES)]
                    fc = brows[n, pl.ds(c * LANES, LANES)]
                    gc = brows[n, pl.ds(R + c * LANES, LANES)]
                    av = jnp.where(pu == 1, oc, ec)
                    bv = jnp.where(pv == 1, gc, fc)
                    q = av * bv if q is None else q + av * bv
                s = jnp.sum(q, axis=0)
                acc = jnp.where(iota == l, acc + s, acc)
            out_v[pl.ds(goff, LANES)] = acc
            return carry

        lax.fori_loop(0, rows_per_round // LANES, group, 0)

    pltpu.sync_copy(out_v, out_hbm.at[pl.ds(base, bpw)])


def kernel(u, it, A, B, bu, bi, mu):
    batch = u.shape[0]
    bpw = batch // NW
    n_chunks = bpw // IDX_CHUNK
    u1 = u.astype(jnp.int32)
    it1 = it.astype(jnp.int32)
    u2 = u1.reshape(NW * n_chunks, IDX_CHUNK)
    it2 = it1.reshape(NW * n_chunks, IDX_CHUNK)
    mu16 = jnp.broadcast_to(jnp.asarray(mu, jnp.float32), (LANES,))

    mesh = plsc.VectorSubcoreMesh(core_axis_name="c", subcore_axis_name="s")

    bias_f = pl.kernel(
        _bias_body,
        out_type=jax.ShapeDtypeStruct((batch,), jnp.float32),
        mesh=mesh,
        compiler_params=pltpu.CompilerParams(
            needs_layout_passes=False, use_tc_tiling_on_sc=False
        ),
        scratch_types=[
            pltpu.VMEM((n_chunks, IDX_CHUNK), jnp.int32),
            pltpu.VMEM((n_chunks, IDX_CHUNK), jnp.int32),
            pltpu.VMEM((bpw,), jnp.float32),
            pltpu.VMEM((bpw,), jnp.float32),
            pltpu.VMEM((LANES,), jnp.float32),
            pltpu.VMEM((bpw,), jnp.float32),
            pltpu.SemaphoreType.DMA,
            pltpu.SemaphoreType.DMA,
        ],
    )
    bs = bias_f(u2, it2, bu, bi, mu16)

    dot_f = pl.kernel(
        _dot_body,
        out_type=jax.ShapeDtypeStruct((batch,), jnp.float32),
        mesh=mesh,
        compiler_params=pltpu.CompilerParams(
            needs_layout_passes=False, use_tc_tiling_on_sc=True
        ),
        scratch_types=[
            pltpu.VMEM((bpw,), jnp.int32),
            pltpu.VMEM((bpw,), jnp.int32),
            pltpu.VMEM((bpw,), jnp.float32),
            pltpu.VMEM((n_chunks, IDX_CHUNK), jnp.int32),
            pltpu.VMEM((n_chunks, IDX_CHUNK), jnp.int32),
            pltpu.VMEM((bpw // 2, IDX_CHUNK), jnp.float32),
            pltpu.VMEM((bpw // 2, IDX_CHUNK), jnp.float32),
            pltpu.VMEM((bpw,), jnp.float32),
            pltpu.SemaphoreType.DMA,
            pltpu.SemaphoreType.DMA,
        ],
    )
    ap = A.reshape(NPAIR, 2 * R)
    bp = B.reshape(NPAIR, 2 * R)
    return dot_f(u1, it1, ap, bp, bs)


# speedup vs baseline: 1.4723x; 1.4723x over previous
import jax
import jax.numpy as jnp
from jax import lax
from jax.experimental import pallas as pl
from jax.experimental.pallas import tpu as pltpu
from jax.experimental.pallas import tpu_sc as plsc

NUM_CORES = 2
NUM_SUBCORES = 16
LANES = 16
NW = NUM_CORES * NUM_SUBCORES
IDX_CHUNK = 128
TILE_ROWS = 8
CHUNK = 32


def _wid():
    return lax.axis_index("s") * NUM_CORES + lax.axis_index("c")


def _bias_body(u_hbm, it_hbm, bu_hbm, bi_hbm, mu_hbm, out_hbm,
               u_v, it_v, buv, biv, mu_v, out_v, sem_bu, sem_bi):
    bpw = out_v.shape[0]
    n_chunks = u_v.shape[0]
    wid = _wid()
    base = wid * bpw

    pltpu.sync_copy(u_hbm.at[pl.ds(wid * n_chunks, n_chunks)], u_v)
    pltpu.sync_copy(it_hbm.at[pl.ds(wid * n_chunks, n_chunks)], it_v)
    pltpu.sync_copy(mu_hbm, mu_v)

    copies = []
    for k in range(n_chunks):
        dst = pl.ds(k * IDX_CHUNK, IDX_CHUNK)
        copies.append(pltpu.async_copy(bu_hbm.at[u_v.at[k]], buv.at[dst], sem_bu))
        copies.append(pltpu.async_copy(bi_hbm.at[it_v.at[k]], biv.at[dst], sem_bi))
    for c in copies:
        c.wait()

    def group(g, carry):
        s = pl.ds(g * LANES, LANES)
        out_v[s] = mu_v[...] + buv[s] + biv[s]
        return carry

    lax.fori_loop(0, bpw // LANES, group, 0)
    pltpu.sync_copy(out_v, out_hbm.at[pl.ds(base, bpw)])


def _factor_body(u_hbm, it_hbm, a_hbm, b_hbm, bs_hbm, out_hbm,
                 u_v, it_v, bs_v, ta, tb, out_v, sem_a, sem_b):
    bpw = out_v.shape[0]
    r = a_hbm.shape[1]
    wid = _wid()
    base = wid * bpw

    pltpu.sync_copy(u_hbm.at[pl.ds(base, bpw)], u_v)
    pltpu.sync_copy(it_hbm.at[pl.ds(base, bpw)], it_v)
    pltpu.sync_copy(bs_hbm.at[pl.ds(base, bpw)], bs_v)

    iota = lax.iota(jnp.int32, LANES)
    n_groups = CHUNK // LANES

    def chunk_body(c, carry):
        uvecs = [u_v[pl.ds(c * CHUNK + h * LANES, LANES)] for h in range(n_groups)]
        ivecs = [it_v[pl.ds(c * CHUNK + h * LANES, LANES)] for h in range(n_groups)]
        for h in range(n_groups):
            for l in range(LANES):
                slot = h * LANES + l
                tu = (uvecs[h][l] >> 3) * TILE_ROWS
                ti = (ivecs[h][l] >> 3) * TILE_ROWS
                pltpu.async_copy(a_hbm.at[pl.ds(tu, TILE_ROWS)], ta.at[slot], sem_a)
                pltpu.async_copy(b_hbm.at[pl.ds(ti, TILE_ROWS)], tb.at[slot], sem_b)
        for slot in range(CHUNK):
            pltpu.make_async_copy(
                a_hbm.at[pl.ds(0, TILE_ROWS)], ta.at[slot], sem_a).wait()
            pltpu.make_async_copy(
                b_hbm.at[pl.ds(0, TILE_ROWS)], tb.at[slot], sem_b).wait()
        for h in range(n_groups):
            acc = bs_v[pl.ds(c * CHUNK + h * LANES, LANES)]
            for l in range(LANES):
                slot = h * LANES + l
                su = uvecs[h][l] & (TILE_ROWS - 1)
                si = ivecs[h][l] & (TILE_ROWS - 1)
                ra = ta.at[slot].at[su]
                rb = tb.at[slot].at[si]
                q = ra[pl.ds(0, LANES)] * rb[pl.ds(0, LANES)]
                for k in range(1, r // LANES):
                    q = q + ra[pl.ds(k * LANES, LANES)] * rb[pl.ds(k * LANES, LANES)]
                s = jnp.sum(q, axis=0)
                acc = jnp.where(iota == l, acc + s, acc)
            out_v[pl.ds(c * CHUNK + h * LANES, LANES)] = acc
        return carry

    lax.fori_loop(0, bpw // CHUNK, chunk_body, 0)
    pltpu.sync_copy(out_v, out_hbm.at[pl.ds(base, bpw)])


def kernel(u, it, A, B, bu, bi, mu):
    batch = u.shape[0]
    r = A.shape[1]
    bpw = batch // NW
    n_chunks = bpw // IDX_CHUNK
    u1 = u.astype(jnp.int32)
    it1 = it.astype(jnp.int32)
    u2 = u1.reshape(NW * n_chunks, IDX_CHUNK)
    it2 = it1.reshape(NW * n_chunks, IDX_CHUNK)
    mu16 = jnp.broadcast_to(jnp.asarray(mu, jnp.float32), (LANES,))

    mesh = plsc.VectorSubcoreMesh(core_axis_name="c", subcore_axis_name="s")
    bias_f = pl.kernel(
        _bias_body,
        out_type=jax.ShapeDtypeStruct((batch,), jnp.float32),
        mesh=mesh,
        compiler_params=pltpu.CompilerParams(
            needs_layout_passes=False, use_tc_tiling_on_sc=False
        ),
        scratch_types=[
            pltpu.VMEM((n_chunks, IDX_CHUNK), jnp.int32),
            pltpu.VMEM((n_chunks, IDX_CHUNK), jnp.int32),
            pltpu.VMEM((bpw,), jnp.float32),
            pltpu.VMEM((bpw,), jnp.float32),
            pltpu.VMEM((LANES,), jnp.float32),
            pltpu.VMEM((bpw,), jnp.float32),
            pltpu.SemaphoreType.DMA,
            pltpu.SemaphoreType.DMA,
        ],
    )
    bs = bias_f(u2, it2, bu, bi, mu16)

    factor_f = pl.kernel(
        _factor_body,
        out_type=jax.ShapeDtypeStruct((batch,), jnp.float32),
        mesh=mesh,
        compiler_params=pltpu.CompilerParams(
            needs_layout_passes=False, use_tc_tiling_on_sc=True
        ),
        scratch_types=[
            pltpu.VMEM((bpw,), jnp.int32),
            pltpu.VMEM((bpw,), jnp.int32),
            pltpu.VMEM((bpw,), jnp.float32),
            pltpu.VMEM((CHUNK, TILE_ROWS, r), jnp.float32),
            pltpu.VMEM((CHUNK, TILE_ROWS, r), jnp.float32),
            pltpu.VMEM((bpw,), jnp.float32),
            pltpu.SemaphoreType.DMA,
            pltpu.SemaphoreType.DMA,
        ],
    )
    return factor_f(u1, it1, A, B, bs)
